# baseline (speedup 1.0000x reference)
import jax
import jax.numpy as jnp
from jax.experimental import pallas as pl
from jax.experimental.pallas import tpu as pltpu

_NTOK = 8192
_DIM = 2048
_NEXP = 64
_CHUNK = 2048
_NCHUNK = _NTOK // _CHUNK
_CBUF = 3
_SUB = 512
_NSUB = _CHUNK // _SUB


def _in_chunk_copy(c, x_hbm, xbuf, insem):
    slot = c % _CBUF
    return pltpu.make_async_copy(
        x_hbm.at[pl.ds(c * _CHUNK, _CHUNK), :], xbuf.at[slot], insem.at[slot])


def _in_sub_wait(c, s, x_hbm, xbuf, insem):
    slot = c % _CBUF
    rows = pl.ds(s * _SUB, _SUB)
    pltpu.make_async_copy(
        x_hbm.at[pl.ds(c * _CHUNK + s * _SUB, _SUB), :],
        xbuf.at[slot, rows, :],
        insem.at[slot],
    ).wait()


def _out_chunk_copy(c, obuf, o_hbm, outsem):
    rows = pl.ds(c * _CHUNK, _CHUNK)
    return pltpu.make_async_copy(
        obuf.at[rows, :], o_hbm.at[rows, :], outsem.at[c])


def _gating_kernel(x_hbm, w_ref, b_ref, o_hbm, xbuf, obuf, insem, outsem):
    for c in range(min(_CBUF, _NCHUNK)):
        _in_chunk_copy(c, x_hbm, xbuf, insem).start()
    for c in range(_NCHUNK):
        slot = c % _CBUF
        for s in range(_NSUB):
            _in_sub_wait(c, s, x_hbm, xbuf, insem)
            xs = xbuf[slot, pl.ds(s * _SUB, _SUB), :]
            logits = jax.lax.dot_general(
                xs, w_ref[...],
                dimension_numbers=(((1,), (1,)), ((), ())),
                preferred_element_type=jnp.float32,
            )
            logits = logits + b_ref[...]
            m = jnp.max(logits, axis=-1, keepdims=True)
            e = jnp.exp(logits - m)
            ssum = jnp.sum(e, axis=-1, keepdims=True)
            obuf[pl.ds(c * _CHUNK + s * _SUB, _SUB), :] = e / ssum
        if c + _CBUF < _NCHUNK:
            _in_chunk_copy(c + _CBUF, x_hbm, xbuf, insem).start()
        _out_chunk_copy(c, obuf, o_hbm, outsem).start(priority=1)
    for c in range(_NCHUNK):
        _out_chunk_copy(c, obuf, o_hbm, outsem).wait()


def kernel(x, W, b):
    n_tokens, input_dim = x.shape
    num_experts = W.shape[0]
    b2 = b.reshape(1, num_experts)
    return pl.pallas_call(
        _gating_kernel,
        in_specs=[
            pl.BlockSpec(memory_space=pltpu.MemorySpace.HBM),
            pl.BlockSpec(memory_space=pltpu.MemorySpace.VMEM),
            pl.BlockSpec(memory_space=pltpu.MemorySpace.VMEM),
        ],
        out_specs=pl.BlockSpec(memory_space=pltpu.MemorySpace.HBM),
        out_shape=jax.ShapeDtypeStruct((n_tokens, num_experts), jnp.float32),
        scratch_shapes=[
            pltpu.VMEM((_CBUF, _CHUNK, _DIM), jnp.float32),
            pltpu.VMEM((_NTOK, _NEXP), jnp.float32),
            pltpu.SemaphoreType.DMA((_CBUF,)),
            pltpu.SemaphoreType.DMA((_NCHUNK,)),
        ],
    )(x, W, b2)


# speedup vs baseline: 1.1208x; 1.1208x over previous
import jax
import jax.numpy as jnp
from jax.experimental import pallas as pl
from jax.experimental.pallas import tpu as pltpu

_NTOK = 8192
_DIM = 2048
_NEXP = 64
_CHUNK = 2048
_NCHUNK = _NTOK // _CHUNK
_CBUF = 2
_SUB = 512
_NSUB = _CHUNK // _SUB


def _in_chunk_copy(c, x_hbm, xbuf, insem):
    slot = c % _CBUF
    return pltpu.make_async_copy(
        x_hbm.at[pl.ds(c * _CHUNK, _CHUNK), :], xbuf.at[slot], insem.at[slot])


def _out_chunk_copy(c, obuf, o_hbm, outsem):
    rows = pl.ds(c * _CHUNK, _CHUNK)
    return pltpu.make_async_copy(
        obuf.at[rows, :], o_hbm.at[rows, :], outsem.at[c])


def _gating_kernel(x_hbm, w_ref, b_ref, o_hbm, xbuf, obuf, insem, outsem):
    for c in range(min(_CBUF, _NCHUNK)):
        _in_chunk_copy(c, x_hbm, xbuf, insem).start()
    for c in range(_NCHUNK):
        slot = c % _CBUF
        _in_chunk_copy(c, x_hbm, xbuf, insem).wait()
        for s in range(_NSUB):
            xs = xbuf[slot, pl.ds(s * _SUB, _SUB), :]
            logits = jax.lax.dot_general(
                xs, w_ref[...],
                dimension_numbers=(((1,), (1,)), ((), ())),
                preferred_element_type=jnp.float32,
            )
            logits = logits + b_ref[...]
            m = jnp.max(logits, axis=-1, keepdims=True)
            e = jnp.exp(logits - m)
            ssum = jnp.sum(e, axis=-1, keepdims=True)
            obuf[pl.ds(c * _CHUNK + s * _SUB, _SUB), :] = e / ssum
        if c + _CBUF < _NCHUNK:
            _in_chunk_copy(c + _CBUF, x_hbm, xbuf, insem).start()
        _out_chunk_copy(c, obuf, o_hbm, outsem).start()
    for c in range(_NCHUNK):
        _out_chunk_copy(c, obuf, o_hbm, outsem).wait()


def kernel(x, W, b):
    n_tokens, input_dim = x.shape
    num_experts = W.shape[0]
    b2 = b.reshape(1, num_experts)
    return pl.pallas_call(
        _gating_kernel,
        in_specs=[
            pl.BlockSpec(memory_space=pltpu.MemorySpace.HBM),
            pl.BlockSpec(memory_space=pltpu.MemorySpace.VMEM),
            pl.BlockSpec(memory_space=pltpu.MemorySpace.VMEM),
        ],
        out_specs=pl.BlockSpec(memory_space=pltpu.MemorySpace.HBM),
        out_shape=jax.ShapeDtypeStruct((n_tokens, num_experts), jnp.float32),
        scratch_shapes=[
            pltpu.VMEM((_CBUF, _CHUNK, _DIM), jnp.float32),
            pltpu.VMEM((_NTOK, _NEXP), jnp.float32),
            pltpu.SemaphoreType.DMA((_CBUF,)),
            pltpu.SemaphoreType.DMA((_NCHUNK,)),
        ],
    )(x, W, b2)


# speedup vs baseline: 1.2063x; 1.0763x over previous
import jax
import jax.numpy as jnp
from jax.experimental import pallas as pl
from jax.experimental.pallas import tpu as pltpu

_TILE = 1024
_NTOK = 8192
_NSTEP = _NTOK // _TILE
_NEXP = 64


def _out_copy(i, obuf, o_hbm, outsem):
    rows = pl.ds(i * _TILE, _TILE)
    return pltpu.make_async_copy(
        obuf.at[rows, :], o_hbm.at[rows, :], outsem.at[i])


def _gating_kernel(x_ref, w_ref, b_ref, o_hbm, obuf, outsem):
    i = pl.program_id(0)
    logits = jax.lax.dot_general(
        x_ref[...], w_ref[...],
        dimension_numbers=(((1,), (1,)), ((), ())),
        preferred_element_type=jnp.float32,
    )
    logits = logits + b_ref[...]
    m = jnp.max(logits, axis=-1, keepdims=True)
    e = jnp.exp(logits - m)
    s = jnp.sum(e, axis=-1, keepdims=True)
    obuf[pl.ds(i * _TILE, _TILE), :] = e / s
    for j in range(_NSTEP):
        @pl.when(i == j)
        def _():
            _out_copy(j, obuf, o_hbm, outsem).start(priority=1)

    @pl.when(i == _NSTEP - 1)
    def _():
        for j in range(_NSTEP):
            _out_copy(j, obuf, o_hbm, outsem).wait()


def kernel(x, W, b):
    n_tokens, input_dim = x.shape
    num_experts = W.shape[0]
    b2 = b.reshape(1, num_experts)
    return pl.pallas_call(
        _gating_kernel,
        grid=(_NSTEP,),
        in_specs=[
            pl.BlockSpec((_TILE, input_dim), lambda i: (i, 0)),
            pl.BlockSpec((num_experts, input_dim), lambda i: (0, 0)),
            pl.BlockSpec((1, num_experts), lambda i: (0, 0)),
        ],
        out_specs=pl.BlockSpec(memory_space=pltpu.MemorySpace.HBM),
        out_shape=jax.ShapeDtypeStruct((n_tokens, num_experts), jnp.float32),
        scratch_shapes=[
            pltpu.VMEM((_NTOK, _NEXP), jnp.float32),
            pltpu.SemaphoreType.DMA((_NSTEP,)),
        ],
        compiler_params=pltpu.CompilerParams(
            dimension_semantics=("arbitrary",),
        ),
    )(x, W, b2)


# speedup vs baseline: 1.2631x; 1.0470x over previous
import jax
import jax.numpy as jnp
from jax.experimental import pallas as pl
from jax.experimental.pallas import tpu as pltpu

_TILE = 1024


def _gating_kernel(x_ref, w_ref, b_ref, out_ref):
    logits = jax.lax.dot_general(
        x_ref[...], w_ref[...],
        dimension_numbers=(((1,), (1,)), ((), ())),
        preferred_element_type=jnp.float32,
    )
    logits = logits + b_ref[...]
    m = jnp.max(logits, axis=-1, keepdims=True)
    e = jnp.exp(logits - m)
    s = jnp.sum(e, axis=-1, keepdims=True)
    out_ref[...] = e / s


def kernel(x, W, b):
    n_tokens, input_dim = x.shape
    num_experts = W.shape[0]
    b2 = b.reshape(1, num_experts)
    return pl.pallas_call(
        _gating_kernel,
        grid=(n_tokens // _TILE,),
        in_specs=[
            pl.BlockSpec((_TILE, input_dim), lambda i: (i, 0)),
            pl.BlockSpec((num_experts, input_dim), lambda i: (0, 0)),
            pl.BlockSpec((1, num_experts), lambda i: (0, 0)),
        ],
        out_specs=pl.BlockSpec((_TILE, num_experts), lambda i: (i, 0)),
        out_shape=jax.ShapeDtypeStruct((n_tokens, num_experts), jnp.float32),
        compiler_params=pltpu.CompilerParams(
            dimension_semantics=("arbitrary",),
        ),
    )(x, W, b2)
